# baseline (speedup 1.0000x reference)
import functools

import jax
import jax.numpy as jnp
from jax import lax
from jax.experimental import pallas as pl
from jax.experimental.pallas import tpu as pltpu
from jax.experimental.pallas import tpu_sc as plsc

N = 10000
E = 160000
F_IN = 10
H = 512
C = 16

NP = 10240
DUMP = N
NTILES = 32
EPT = E // NTILES
EB = 128
NJ = 5120 // EB
RPT = NP // 16
GB = 2
NEG = 0.01

_mesh = plsc.VectorSubcoreMesh(core_axis_name="c", subcore_axis_name="s")
_sc_params = pltpu.CompilerParams(use_tc_tiling_on_sc=False)


def _fill_zeros(ref, nrows, width):
    def body(i, _):
        for l in range(width // 16):
            ref[i, pl.ds(l * 16, 16)] = jnp.zeros((16,), jnp.float32)
        return 0
    lax.fori_loop(0, nrows, body, 0)


@functools.partial(
    pl.kernel,
    out_type=jax.ShapeDtypeStruct((2, NP, 16), jnp.float32),
    mesh=_mesh,
    compiler_params=_sc_params,
    scratch_types=[
        pltpu.VMEM((NJ, EB), jnp.int32),
        pltpu.VMEM((EB, 16), jnp.float32),
        pltpu.VMEM((RPT, 16), jnp.float32),
        pltpu.VMEM_SHARED((NP, 16), jnp.float32),
    ],
)
def _sc_degree(dstp_hbm, deg_out, dst_v, ones_v, stage_v, acc_sh):
    c = lax.axis_index("c")
    s = lax.axis_index("s")
    w = c * 16 + s
    pltpu.sync_copy(dstp_hbm.at[w], dst_v)

    def fill_ones(i, _):
        ones_v[i, :] = jnp.ones((16,), jnp.float32)
        return 0
    lax.fori_loop(0, EB, fill_ones, 0)
    _fill_zeros(stage_v, RPT, 16)
    pltpu.sync_copy(stage_v, acc_sh.at[pl.ds(s * RPT, RPT)])
    plsc.subcore_barrier()

    def body(j, _):
        pltpu.sync_copy(ones_v, acc_sh.at[dst_v.at[j]], add=True)
        return 0
    lax.fori_loop(0, NJ, body, 0)
    plsc.subcore_barrier()

    pltpu.sync_copy(acc_sh.at[pl.ds(s * RPT, RPT)],
                    deg_out.at[c, pl.ds(s * RPT, RPT)])


@functools.partial(
    pl.kernel,
    out_type=jax.ShapeDtypeStruct((2, NP, 16), jnp.float32),
    mesh=_mesh,
    compiler_params=_sc_params,
    scratch_types=[
        pltpu.VMEM((NJ, EB), jnp.int32),
        pltpu.VMEM((NJ, EB), jnp.int32),
        pltpu.VMEM((EB, 16), jnp.float32),
        pltpu.VMEM((RPT, 16), jnp.float32),
        pltpu.VMEM_SHARED((NP, 16), jnp.float32),
        pltpu.SemaphoreType.DMA,
    ],
)
def _sc_agg16(xs_hbm, srcp_hbm, dstp_hbm, s_out,
              src_v, dst_v, rows_v, stage_v, acc_sh, sem):
    c = lax.axis_index("c")
    s = lax.axis_index("s")
    w = c * 16 + s
    pltpu.sync_copy(srcp_hbm.at[w], src_v)
    pltpu.sync_copy(dstp_hbm.at[w], dst_v)
    _fill_zeros(stage_v, RPT, 16)
    pltpu.sync_copy(stage_v, acc_sh.at[pl.ds(s * RPT, RPT)])
    plsc.subcore_barrier()

    def body(j, _):
        pltpu.async_copy(xs_hbm.at[src_v.at[j]], rows_v, sem).wait()
        pltpu.sync_copy(rows_v, acc_sh.at[dst_v.at[j]], add=True)
        return 0
    lax.fori_loop(0, NJ, body, 0)
    plsc.subcore_barrier()

    pltpu.sync_copy(acc_sh.at[pl.ds(s * RPT, RPT)],
                    s_out.at[c, pl.ds(s * RPT, RPT)])


@functools.partial(
    pl.kernel,
    out_type=jax.ShapeDtypeStruct((2, 4 * NP, 128), jnp.float32),
    mesh=_mesh,
    compiler_params=_sc_params,
    scratch_types=[
        pltpu.VMEM((NJ * 2, EB // 2), jnp.int32),
        pltpu.VMEM((NJ * 2, EB // 2), jnp.int32),
        pltpu.VMEM((EB // 2, 128), jnp.float32),
        pltpu.VMEM((EB // 2, 128), jnp.float32),
        pltpu.VMEM((EB // 2, 128), jnp.float32),
        pltpu.VMEM((EB // 2, 128), jnp.float32),
        pltpu.VMEM((32, 128), jnp.float32),
        pltpu.VMEM_SHARED((NP, 128), jnp.float32),
        pltpu.SemaphoreType.DMA,
        pltpu.SemaphoreType.DMA,
        pltpu.SemaphoreType.DMA,
        pltpu.SemaphoreType.DMA,
    ],
)
def _sc_agg128(zsf_hbm, srcp_all_hbm, dstp_hbm, s_out,
               src_v, dst_v, r0, r1, r2, r3, zero_v, acc_sh,
               g0, g1, g2, g3):
    rows = (r0, r1, r2, r3)
    gsem = (g0, g1, g2, g3)
    c = lax.axis_index("c")
    s = lax.axis_index("s")
    w = c * 16 + s
    pltpu.sync_copy(dstp_hbm.at[w], dst_v)
    _fill_zeros(zero_v, 32, 128)

    for k in range(4):
        pltpu.sync_copy(srcp_all_hbm.at[k, w], src_v)
        for q in range(RPT // 32):
            pltpu.sync_copy(zero_v, acc_sh.at[pl.ds(s * RPT + q * 32, 32)])
        plsc.subcore_barrier()

        for r in range(3):
            pltpu.async_copy(zsf_hbm.at[src_v.at[r]], rows[r], gsem[r])

        def body(g, _):
            j = 4 * g
            pltpu.async_copy(zsf_hbm.at[src_v.at[j + 3]], rows[3], gsem[3])
            for r in range(4):
                pltpu.make_async_copy(
                    zsf_hbm.at[src_v.at[j + r]], rows[r], gsem[r]).wait()
                pltpu.sync_copy(rows[r], acc_sh.at[dst_v.at[j + r]],
                                add=True)
                if r < 3:
                    @pl.when(g + 1 < (NJ * 2) // 4)
                    def _():
                        pltpu.async_copy(zsf_hbm.at[src_v.at[j + 4 + r]],
                                         rows[r], gsem[r])
            return 0
        lax.fori_loop(0, (NJ * 2) // 4, body, 0)
        plsc.subcore_barrier()

        pltpu.sync_copy(acc_sh.at[pl.ds(s * RPT, RPT)],
                        s_out.at[c, pl.ds(k * NP + s * RPT, RPT)])


def _prep_body(degp_ref, xp_ref, dinv_ref, xs_ref):
    deg = degp_ref[0, :, 0:1] + degp_ref[1, :, 0:1] + 1.0
    dinv = lax.rsqrt(deg)
    dinv_ref[...] = dinv
    xs_ref[...] = xp_ref[...] * dinv


def _tc_prep(degp, xp):
    return pl.pallas_call(
        _prep_body,
        out_shape=(
            jax.ShapeDtypeStruct((NP, 1), jnp.float32),
            jax.ShapeDtypeStruct((NP, 16), jnp.float32),
        ),
    )(degp, xp)


def _l1_body(s16_ref, xs_ref, dinv_ref, w_ref, b_ref, out_ref):
    dinv = dinv_ref[...]
    u = dinv * (s16_ref[0] + s16_ref[1] + xs_ref[...])
    z = jnp.dot(u, w_ref[...], preferred_element_type=jnp.float32)
    z = z + b_ref[...]
    h = jnp.where(z >= 0, z, NEG * z)
    out_ref[...] = h * dinv


def _tc_layer1(s16, xs, dinv, w1p, b1):
    bm = 1024
    grid = (NP // bm,)
    return pl.pallas_call(
        _l1_body,
        grid=grid,
        in_specs=[
            pl.BlockSpec((2, bm, 16), lambda i: (0, i, 0)),
            pl.BlockSpec((bm, 16), lambda i: (i, 0)),
            pl.BlockSpec((bm, 1), lambda i: (i, 0)),
            pl.BlockSpec((16, H), lambda i: (0, 0)),
            pl.BlockSpec((1, H), lambda i: (0, 0)),
        ],
        out_specs=pl.BlockSpec((bm, H), lambda i: (i, 0)),
        out_shape=jax.ShapeDtypeStruct((NP, H), jnp.float32),
        compiler_params=pltpu.CompilerParams(
            dimension_semantics=("parallel",)),
    )(s16, xs, dinv, w1p, b1)


def _mm_body(hd_ref, w_ref, zs_ref):
    zs_ref[0] = jnp.dot(hd_ref[...], w_ref[...],
                        preferred_element_type=jnp.float32)


def _tc_matmul_chunked(hd, w):
    bm = 512
    grid = (NP // bm, 4)
    return pl.pallas_call(
        _mm_body,
        grid=grid,
        in_specs=[
            pl.BlockSpec((bm, H), lambda i, j: (i, 0)),
            pl.BlockSpec((H, 128), lambda i, j: (0, j)),
        ],
        out_specs=pl.BlockSpec((1, bm, 128), lambda i, j: (j, i, 0)),
        out_shape=jax.ShapeDtypeStruct((4, NP, 128), jnp.float32),
        compiler_params=pltpu.CompilerParams(
            dimension_semantics=("parallel", "parallel")),
    )(hd, w)


def _ew_body(scale_out, sp_ref, zs_ref, dinv_ref, b_ref, out_ref):
    dinv = dinv_ref[...]
    z = dinv * (sp_ref[0, 0] + sp_ref[1, 0] + zs_ref[0]) + b_ref[0]
    h = jnp.where(z >= 0, z, NEG * z)
    out_ref[...] = h * dinv if scale_out else h


def _tc_ew(sp, zs, dinv, b4, scale_out):
    bm = 1024
    grid = (NP // bm, 4)
    return pl.pallas_call(
        functools.partial(_ew_body, scale_out),
        grid=grid,
        in_specs=[
            pl.BlockSpec((2, 1, bm, 128), lambda i, j: (0, j, i, 0)),
            pl.BlockSpec((1, bm, 128), lambda i, j: (j, i, 0)),
            pl.BlockSpec((bm, 1), lambda i, j: (i, 0)),
            pl.BlockSpec((1, 1, 128), lambda i, j: (j, 0, 0)),
        ],
        out_specs=pl.BlockSpec((bm, 128), lambda i, j: (i, j)),
        out_shape=jax.ShapeDtypeStruct((NP, H), jnp.float32),
        compiler_params=pltpu.CompilerParams(
            dimension_semantics=("parallel", "parallel")),
    )(sp, zs, dinv, b4)


def _fc_body(h_ref, w_ref, b_ref, out_ref):
    out_ref[...] = jnp.dot(h_ref[...], w_ref[...],
                           preferred_element_type=jnp.float32) + b_ref[...]


def _tc_fc(h, wfc, bfc2):
    bm = 1024
    grid = (NP // bm,)
    return pl.pallas_call(
        _fc_body,
        grid=grid,
        in_specs=[
            pl.BlockSpec((bm, H), lambda i: (i, 0)),
            pl.BlockSpec((H, C), lambda i: (0, 0)),
            pl.BlockSpec((1, C), lambda i: (0, 0)),
        ],
        out_specs=pl.BlockSpec((bm, C), lambda i: (i, 0)),
        out_shape=jax.ShapeDtypeStruct((NP, C), jnp.float32),
        compiler_params=pltpu.CompilerParams(
            dimension_semantics=("parallel",)),
    )(h, wfc, bfc2)


def kernel(x, edge_index, W1, b1, W2, b2, W3, b3, Wfc, bfc):
    order = jnp.argsort(edge_index[0])
    src = edge_index[0][order].astype(jnp.int32).reshape(NTILES, EPT)
    dst = edge_index[1][order].astype(jnp.int32).reshape(NTILES, EPT)
    pad = NJ * EB - EPT
    srcp = jnp.concatenate(
        [src, jnp.zeros((NTILES, pad), jnp.int32)], axis=1
    ).reshape(NTILES, NJ, EB)
    dstp = jnp.concatenate(
        [dst, jnp.full((NTILES, pad), DUMP, jnp.int32)], axis=1
    ).reshape(NTILES, NJ, EB)

    srcp_all = (srcp[None] +
                (jnp.arange(4, dtype=jnp.int32) * NP)[:, None, None, None])

    xp = jnp.zeros((NP, 16), jnp.float32).at[:N, :F_IN].set(x)
    w1p = jnp.zeros((16, H), jnp.float32).at[:F_IN].set(W1)


    degp = _sc_degree(dstp)
    dinv, xs = _tc_prep(degp, xp)
    s16 = _sc_agg16(xs, srcp, dstp)
    hd1 = _tc_layer1(s16, xs, dinv, w1p, b1.reshape(1, H))

    zs2 = _tc_matmul_chunked(hd1, W2)
    srcp_h = srcp_all.reshape(4, NTILES, NJ * 2, EB // 2)
    dstp_h = dstp.reshape(NTILES, NJ * 2, EB // 2)
    sp2 = _sc_agg128(zs2.reshape(4 * NP, 128), srcp_h, dstp_h)
    hd2 = _tc_ew(sp2.reshape(2, 4, NP, 128), zs2, dinv,
                 b2.reshape(4, 1, 128), True)

    zs3 = _tc_matmul_chunked(hd2, W3)
    sp3 = _sc_agg128(zs3.reshape(4 * NP, 128), srcp_h, dstp_h)
    h3 = _tc_ew(sp3.reshape(2, 4, NP, 128), zs3, dinv,
                b3.reshape(4, 1, 128), False)

    out = _tc_fc(h3, Wfc, bfc.reshape(1, C))
    return out[:N]


# speedup vs baseline: 1.7974x; 1.7974x over previous
import functools

import jax
import jax.numpy as jnp
from jax import lax
from jax.experimental import pallas as pl
from jax.experimental.pallas import tpu as pltpu
from jax.experimental.pallas import tpu_sc as plsc

N = 10000
E = 160000
F_IN = 10
H = 512
C = 16

NP = 10240
DUMP = N
NTILES = 32
EPT = E // NTILES
EB = 128
NJ = 5120 // EB
RPT = NP // 16
GB = 2
NEG = 0.01

_mesh = plsc.VectorSubcoreMesh(core_axis_name="c", subcore_axis_name="s")
_sc_params = pltpu.CompilerParams(use_tc_tiling_on_sc=False)


def _fill_zeros(ref, nrows, width):
    def body(i, _):
        for l in range(width // 16):
            ref[i, pl.ds(l * 16, 16)] = jnp.zeros((16,), jnp.float32)
        return 0
    lax.fori_loop(0, nrows, body, 0)


@functools.partial(
    pl.kernel,
    out_type=jax.ShapeDtypeStruct((2, NP, 16), jnp.float32),
    mesh=_mesh,
    compiler_params=_sc_params,
    scratch_types=[
        pltpu.VMEM((NJ, EB), jnp.int32),
        pltpu.VMEM((EB, 16), jnp.float32),
        pltpu.VMEM((RPT, 16), jnp.float32),
        pltpu.VMEM_SHARED((NP, 16), jnp.float32),
    ],
)
def _sc_degree(dstp_hbm, deg_out, dst_v, ones_v, stage_v, acc_sh):
    c = lax.axis_index("c")
    s = lax.axis_index("s")
    w = c * 16 + s
    pltpu.sync_copy(dstp_hbm.at[w], dst_v)

    def fill_ones(i, _):
        ones_v[i, :] = jnp.ones((16,), jnp.float32)
        return 0
    lax.fori_loop(0, EB, fill_ones, 0)
    _fill_zeros(stage_v, RPT, 16)
    pltpu.sync_copy(stage_v, acc_sh.at[pl.ds(s * RPT, RPT)])
    plsc.subcore_barrier()

    def body(j, _):
        pltpu.sync_copy(ones_v, acc_sh.at[dst_v.at[j]], add=True)
        return 0
    lax.fori_loop(0, NJ, body, 0)
    plsc.subcore_barrier()

    pltpu.sync_copy(acc_sh.at[pl.ds(s * RPT, RPT)],
                    deg_out.at[c, pl.ds(s * RPT, RPT)])


@functools.partial(
    pl.kernel,
    out_type=jax.ShapeDtypeStruct((2, NP, 16), jnp.float32),
    mesh=_mesh,
    compiler_params=_sc_params,
    scratch_types=[
        pltpu.VMEM((NJ, EB), jnp.int32),
        pltpu.VMEM((NJ, EB), jnp.int32),
        pltpu.VMEM((EB, 16), jnp.float32),
        pltpu.VMEM((RPT, 16), jnp.float32),
        pltpu.VMEM_SHARED((NP, 16), jnp.float32),
        pltpu.SemaphoreType.DMA,
    ],
)
def _sc_agg16(xs_hbm, srcp_hbm, dstp_hbm, s_out,
              src_v, dst_v, rows_v, stage_v, acc_sh, sem):
    c = lax.axis_index("c")
    s = lax.axis_index("s")
    w = c * 16 + s
    pltpu.sync_copy(srcp_hbm.at[w], src_v)
    pltpu.sync_copy(dstp_hbm.at[w], dst_v)
    _fill_zeros(stage_v, RPT, 16)
    pltpu.sync_copy(stage_v, acc_sh.at[pl.ds(s * RPT, RPT)])
    plsc.subcore_barrier()

    def body(j, _):
        pltpu.async_copy(xs_hbm.at[src_v.at[j]], rows_v, sem).wait()
        pltpu.sync_copy(rows_v, acc_sh.at[dst_v.at[j]], add=True)
        return 0
    lax.fori_loop(0, NJ, body, 0)
    plsc.subcore_barrier()

    pltpu.sync_copy(acc_sh.at[pl.ds(s * RPT, RPT)],
                    s_out.at[c, pl.ds(s * RPT, RPT)])


@functools.partial(
    pl.kernel,
    out_type=jax.ShapeDtypeStruct((2, 8 * NP, 64), jnp.float32),
    mesh=_mesh,
    compiler_params=_sc_params,
    scratch_types=[
        pltpu.VMEM((NJ, EB), jnp.int32),
        pltpu.VMEM((NJ, EB), jnp.int32),
        pltpu.VMEM((EB, 64), jnp.float32),
        pltpu.VMEM((EB, 64), jnp.float32),
        pltpu.VMEM((32, 64), jnp.float32),
        pltpu.VMEM_SHARED((NP, 64), jnp.float32),
        pltpu.VMEM_SHARED((NP, 64), jnp.float32),
        pltpu.SemaphoreType.DMA,
        pltpu.SemaphoreType.DMA,
    ],
)
def _sc_agg64(zsf_hbm, srcp_hbm, dstp_hbm, s_out,
              src_v, dst_v, rows0, rows1, zero_v, tab_sh, acc_sh,
              gsem0, gsem1):
    c = lax.axis_index("c")
    s = lax.axis_index("s")
    w = c * 16 + s
    pltpu.sync_copy(srcp_hbm.at[w], src_v)
    pltpu.sync_copy(dstp_hbm.at[w], dst_v)
    _fill_zeros(zero_v, 32, 64)

    for k in range(8):
        pltpu.sync_copy(zsf_hbm.at[pl.ds(k * NP + s * RPT, RPT)],
                        tab_sh.at[pl.ds(s * RPT, RPT)])
        for q in range(RPT // 32):
            pltpu.sync_copy(zero_v, acc_sh.at[pl.ds(s * RPT + q * 32, 32)])
        plsc.subcore_barrier()

        pltpu.async_copy(tab_sh.at[src_v.at[0]], rows0, gsem0)

        def body(g, _):
            j0 = 2 * g
            pltpu.async_copy(tab_sh.at[src_v.at[j0 + 1]], rows1, gsem1)
            pltpu.make_async_copy(
                tab_sh.at[src_v.at[j0]], rows0, gsem0).wait()
            pltpu.sync_copy(rows0, acc_sh.at[dst_v.at[j0]], add=True)

            @pl.when(g + 1 < NJ // 2)
            def _():
                pltpu.async_copy(tab_sh.at[src_v.at[j0 + 2]], rows0, gsem0)
            pltpu.make_async_copy(
                tab_sh.at[src_v.at[j0 + 1]], rows1, gsem1).wait()
            pltpu.sync_copy(rows1, acc_sh.at[dst_v.at[j0 + 1]], add=True)
            return 0
        lax.fori_loop(0, NJ // 2, body, 0)
        plsc.subcore_barrier()

        pltpu.sync_copy(acc_sh.at[pl.ds(s * RPT, RPT)],
                        s_out.at[c, pl.ds(k * NP + s * RPT, RPT)])


def _prep_body(degp_ref, xp_ref, dinv_ref, xs_ref):
    deg = degp_ref[0, :, 0:1] + degp_ref[1, :, 0:1] + 1.0
    dinv = lax.rsqrt(deg)
    dinv_ref[...] = dinv
    xs_ref[...] = xp_ref[...] * dinv


def _tc_prep(degp, xp):
    return pl.pallas_call(
        _prep_body,
        out_shape=(
            jax.ShapeDtypeStruct((NP, 1), jnp.float32),
            jax.ShapeDtypeStruct((NP, 16), jnp.float32),
        ),
    )(degp, xp)


def _l1_body(s16_ref, xs_ref, dinv_ref, w_ref, b_ref, out_ref):
    dinv = dinv_ref[...]
    u = dinv * (s16_ref[0] + s16_ref[1] + xs_ref[...])
    z = jnp.dot(u, w_ref[...], preferred_element_type=jnp.float32)
    z = z + b_ref[...]
    h = jnp.where(z >= 0, z, NEG * z)
    out_ref[...] = h * dinv


def _tc_layer1(s16, xs, dinv, w1p, b1):
    bm = 1024
    grid = (NP // bm,)
    return pl.pallas_call(
        _l1_body,
        grid=grid,
        in_specs=[
            pl.BlockSpec((2, bm, 16), lambda i: (0, i, 0)),
            pl.BlockSpec((bm, 16), lambda i: (i, 0)),
            pl.BlockSpec((bm, 1), lambda i: (i, 0)),
            pl.BlockSpec((16, H), lambda i: (0, 0)),
            pl.BlockSpec((1, H), lambda i: (0, 0)),
        ],
        out_specs=pl.BlockSpec((bm, H), lambda i: (i, 0)),
        out_shape=jax.ShapeDtypeStruct((NP, H), jnp.float32),
        compiler_params=pltpu.CompilerParams(
            dimension_semantics=("parallel",)),
    )(s16, xs, dinv, w1p, b1)


def _mm_body(hd_ref, w_ref, zs_ref):
    zs_ref[0] = jnp.dot(hd_ref[...], w_ref[...],
                        preferred_element_type=jnp.float32)


def _tc_matmul_chunked(hd, w):
    bm = 512
    grid = (NP // bm, 4)
    return pl.pallas_call(
        _mm_body,
        grid=grid,
        in_specs=[
            pl.BlockSpec((bm, H), lambda i, j: (i, 0)),
            pl.BlockSpec((H, 128), lambda i, j: (0, j)),
        ],
        out_specs=pl.BlockSpec((1, bm, 128), lambda i, j: (j, i, 0)),
        out_shape=jax.ShapeDtypeStruct((4, NP, 128), jnp.float32),
        compiler_params=pltpu.CompilerParams(
            dimension_semantics=("parallel", "parallel")),
    )(hd, w)


def _ew_body(scale_out, sp_ref, zs_ref, dinv_ref, b_ref, out_ref):
    dinv = dinv_ref[...]
    s_lo = sp_ref[0, 0] + sp_ref[1, 0]
    s_hi = sp_ref[0, 1] + sp_ref[1, 1]
    z = dinv * (jnp.concatenate([s_lo, s_hi], axis=1) + zs_ref[0]) + b_ref[0]
    h = jnp.where(z >= 0, z, NEG * z)
    out_ref[...] = h * dinv if scale_out else h


def _tc_ew(sp, zs, dinv, b4, scale_out):
    bm = 1024
    grid = (NP // bm, 4)
    return pl.pallas_call(
        functools.partial(_ew_body, scale_out),
        grid=grid,
        in_specs=[
            pl.BlockSpec((2, 2, bm, 64), lambda i, j: (0, j, i, 0)),
            pl.BlockSpec((1, bm, 128), lambda i, j: (j, i, 0)),
            pl.BlockSpec((bm, 1), lambda i, j: (i, 0)),
            pl.BlockSpec((1, 1, 128), lambda i, j: (j, 0, 0)),
        ],
        out_specs=pl.BlockSpec((bm, 128), lambda i, j: (i, j)),
        out_shape=jax.ShapeDtypeStruct((NP, H), jnp.float32),
        compiler_params=pltpu.CompilerParams(
            dimension_semantics=("parallel", "parallel")),
    )(sp, zs, dinv, b4)


def _fc_body(h_ref, w_ref, b_ref, out_ref):
    out_ref[...] = jnp.dot(h_ref[...], w_ref[...],
                           preferred_element_type=jnp.float32) + b_ref[...]


def _tc_fc(h, wfc, bfc2):
    bm = 1024
    grid = (NP // bm,)
    return pl.pallas_call(
        _fc_body,
        grid=grid,
        in_specs=[
            pl.BlockSpec((bm, H), lambda i: (i, 0)),
            pl.BlockSpec((H, C), lambda i: (0, 0)),
            pl.BlockSpec((1, C), lambda i: (0, 0)),
        ],
        out_specs=pl.BlockSpec((bm, C), lambda i: (i, 0)),
        out_shape=jax.ShapeDtypeStruct((NP, C), jnp.float32),
        compiler_params=pltpu.CompilerParams(
            dimension_semantics=("parallel",)),
    )(h, wfc, bfc2)


def kernel(x, edge_index, W1, b1, W2, b2, W3, b3, Wfc, bfc):
    src = edge_index[0].astype(jnp.int32).reshape(NTILES, EPT)
    dst = edge_index[1].astype(jnp.int32).reshape(NTILES, EPT)
    pad = NJ * EB - EPT
    srcp = jnp.concatenate(
        [src, jnp.zeros((NTILES, pad), jnp.int32)], axis=1
    ).reshape(NTILES, NJ, EB)
    dstp = jnp.concatenate(
        [dst, jnp.full((NTILES, pad), DUMP, jnp.int32)], axis=1
    ).reshape(NTILES, NJ, EB)

    xp = jnp.zeros((NP, 16), jnp.float32).at[:N, :F_IN].set(x)
    w1p = jnp.zeros((16, H), jnp.float32).at[:F_IN].set(W1)


    degp = _sc_degree(dstp)
    dinv, xs = _tc_prep(degp, xp)
    s16 = _sc_agg16(xs, srcp, dstp)
    hd1 = _tc_layer1(s16, xs, dinv, w1p, b1.reshape(1, H))

    def chunk64(zs):
        return zs.reshape(4, NP, 2, 64).transpose(0, 2, 1, 3).reshape(
            8 * NP, 64)

    zs2 = _tc_matmul_chunked(hd1, W2)
    sp2 = _sc_agg64(chunk64(zs2), srcp, dstp)
    hd2 = _tc_ew(sp2.reshape(2, 8, NP, 64), zs2, dinv,
                 b2.reshape(4, 1, 128), True)

    zs3 = _tc_matmul_chunked(hd2, W3)
    sp3 = _sc_agg64(chunk64(zs3), srcp, dstp)
    h3 = _tc_ew(sp3.reshape(2, 8, NP, 64), zs3, dinv,
                b3.reshape(4, 1, 128), False)

    out = _tc_fc(h3, Wfc, bfc.reshape(1, C))
    return out[:N]


# speedup vs baseline: 2.2754x; 1.2659x over previous
import functools

import jax
import jax.numpy as jnp
from jax import lax
from jax.experimental import pallas as pl
from jax.experimental.pallas import tpu as pltpu
from jax.experimental.pallas import tpu_sc as plsc

N = 10000
E = 160000
F_IN = 10
H = 512
C = 16

NP = 10240
DUMP = N
NTILES = 32
EPT = E // NTILES
EB = 128
NJ = 5120 // EB
RPT = NP // 16
GB = 2
NEG = 0.01

_mesh = plsc.VectorSubcoreMesh(core_axis_name="c", subcore_axis_name="s")
_sc_params = pltpu.CompilerParams(use_tc_tiling_on_sc=False)


def _fill_zeros(ref, nrows, width):
    def body(i, _):
        for l in range(width // 16):
            ref[i, pl.ds(l * 16, 16)] = jnp.zeros((16,), jnp.float32)
        return 0
    lax.fori_loop(0, nrows, body, 0)


@functools.partial(
    pl.kernel,
    out_type=jax.ShapeDtypeStruct((2, NP, 16), jnp.float32),
    mesh=_mesh,
    compiler_params=_sc_params,
    scratch_types=[
        pltpu.VMEM((NJ, EB), jnp.int32),
        pltpu.VMEM((EB, 16), jnp.float32),
        pltpu.VMEM((RPT, 16), jnp.float32),
        pltpu.VMEM_SHARED((NP, 16), jnp.float32),
    ],
)
def _sc_degree(dstp_hbm, deg_out, dst_v, ones_v, stage_v, acc_sh):
    c = lax.axis_index("c")
    s = lax.axis_index("s")
    w = c * 16 + s
    pltpu.sync_copy(dstp_hbm.at[w], dst_v)

    def fill_ones(i, _):
        ones_v[i, :] = jnp.ones((16,), jnp.float32)
        return 0
    lax.fori_loop(0, EB, fill_ones, 0)
    _fill_zeros(stage_v, RPT, 16)
    pltpu.sync_copy(stage_v, acc_sh.at[pl.ds(s * RPT, RPT)])
    plsc.subcore_barrier()

    def body(j, _):
        pltpu.sync_copy(ones_v, acc_sh.at[dst_v.at[j]], add=True)
        return 0
    lax.fori_loop(0, NJ, body, 0)
    plsc.subcore_barrier()

    pltpu.sync_copy(acc_sh.at[pl.ds(s * RPT, RPT)],
                    deg_out.at[c, pl.ds(s * RPT, RPT)])


@functools.partial(
    pl.kernel,
    out_type=jax.ShapeDtypeStruct((2, NP, 16), jnp.float32),
    mesh=_mesh,
    compiler_params=_sc_params,
    scratch_types=[
        pltpu.VMEM((NJ, EB), jnp.int32),
        pltpu.VMEM((NJ, EB), jnp.int32),
        pltpu.VMEM((EB, 16), jnp.float32),
        pltpu.VMEM((RPT, 16), jnp.float32),
        pltpu.VMEM_SHARED((NP, 16), jnp.float32),
        pltpu.SemaphoreType.DMA,
    ],
)
def _sc_agg16(xs_hbm, srcp_hbm, dstp_hbm, s_out,
              src_v, dst_v, rows_v, stage_v, acc_sh, sem):
    c = lax.axis_index("c")
    s = lax.axis_index("s")
    w = c * 16 + s
    pltpu.sync_copy(srcp_hbm.at[w], src_v)
    pltpu.sync_copy(dstp_hbm.at[w], dst_v)
    _fill_zeros(stage_v, RPT, 16)
    pltpu.sync_copy(stage_v, acc_sh.at[pl.ds(s * RPT, RPT)])
    plsc.subcore_barrier()

    def body(j, _):
        pltpu.async_copy(xs_hbm.at[src_v.at[j]], rows_v, sem).wait()
        pltpu.sync_copy(rows_v, acc_sh.at[dst_v.at[j]], add=True)
        return 0
    lax.fori_loop(0, NJ, body, 0)
    plsc.subcore_barrier()

    pltpu.sync_copy(acc_sh.at[pl.ds(s * RPT, RPT)],
                    s_out.at[c, pl.ds(s * RPT, RPT)])


@functools.partial(
    pl.kernel,
    out_type=jax.ShapeDtypeStruct((2, 4 * NP, 128), jnp.float32),
    mesh=_mesh,
    compiler_params=_sc_params,
    scratch_types=[
        pltpu.VMEM((NJ, EB), jnp.int32),
        pltpu.VMEM((NJ, EB), jnp.int32),
        pltpu.VMEM((EB, 64), jnp.float32),
        pltpu.VMEM((EB, 64), jnp.float32),
        pltpu.VMEM((32, 64), jnp.float32),
        pltpu.VMEM_SHARED((NP, 64), jnp.float32),
        pltpu.VMEM_SHARED((NP, 64), jnp.float32),
        pltpu.SemaphoreType.DMA,
        pltpu.SemaphoreType.DMA,
    ],
)
def _sc_agg64(zsf_hbm, srcp_hbm, dstp_hbm, s_out,
              src_v, dst_v, rows0, rows1, zero_v, tab_sh, acc_sh,
              gsem0, gsem1):
    c = lax.axis_index("c")
    s = lax.axis_index("s")
    w = c * 16 + s
    pltpu.sync_copy(srcp_hbm.at[w], src_v)
    pltpu.sync_copy(dstp_hbm.at[w], dst_v)
    _fill_zeros(zero_v, 32, 64)

    for k in range(8):
        k128, h = k // 2, k % 2
        pltpu.sync_copy(
            zsf_hbm.at[pl.ds(k128 * NP + s * RPT, RPT), pl.ds(h * 64, 64)],
            tab_sh.at[pl.ds(s * RPT, RPT)])
        for q in range(RPT // 32):
            pltpu.sync_copy(zero_v, acc_sh.at[pl.ds(s * RPT + q * 32, 32)])
        plsc.subcore_barrier()

        pltpu.async_copy(tab_sh.at[src_v.at[0]], rows0, gsem0)

        def body(g, _):
            j0 = 2 * g
            pltpu.async_copy(tab_sh.at[src_v.at[j0 + 1]], rows1, gsem1)
            pltpu.make_async_copy(
                tab_sh.at[src_v.at[j0]], rows0, gsem0).wait()
            pltpu.sync_copy(rows0, acc_sh.at[dst_v.at[j0]], add=True)

            @pl.when(g + 1 < NJ // 2)
            def _():
                pltpu.async_copy(tab_sh.at[src_v.at[j0 + 2]], rows0, gsem0)
            pltpu.make_async_copy(
                tab_sh.at[src_v.at[j0 + 1]], rows1, gsem1).wait()
            pltpu.sync_copy(rows1, acc_sh.at[dst_v.at[j0 + 1]], add=True)
            return 0
        lax.fori_loop(0, NJ // 2, body, 0)
        plsc.subcore_barrier()

        pltpu.sync_copy(
            acc_sh.at[pl.ds(s * RPT, RPT)],
            s_out.at[c, pl.ds(k128 * NP + s * RPT, RPT), pl.ds(h * 64, 64)])


def _prep_body(degp_ref, xp_ref, dinv_ref, xs_ref):
    deg = degp_ref[0, :, 0:1] + degp_ref[1, :, 0:1] + 1.0
    dinv = lax.rsqrt(deg)
    dinv_ref[...] = dinv
    xs_ref[...] = xp_ref[...] * dinv


def _tc_prep(degp, xp):
    return pl.pallas_call(
        _prep_body,
        out_shape=(
            jax.ShapeDtypeStruct((NP, 1), jnp.float32),
            jax.ShapeDtypeStruct((NP, 16), jnp.float32),
        ),
    )(degp, xp)


def _l1_body(s16_ref, xs_ref, dinv_ref, w_ref, b_ref, out_ref):
    dinv = dinv_ref[...]
    u = dinv * (s16_ref[0] + s16_ref[1] + xs_ref[...])
    z = jnp.dot(u, w_ref[...], preferred_element_type=jnp.float32)
    z = z + b_ref[...]
    h = jnp.where(z >= 0, z, NEG * z)
    out_ref[...] = h * dinv


def _tc_layer1(s16, xs, dinv, w1p, b1):
    bm = 1024
    grid = (NP // bm,)
    return pl.pallas_call(
        _l1_body,
        grid=grid,
        in_specs=[
            pl.BlockSpec((2, bm, 16), lambda i: (0, i, 0)),
            pl.BlockSpec((bm, 16), lambda i: (i, 0)),
            pl.BlockSpec((bm, 1), lambda i: (i, 0)),
            pl.BlockSpec((16, H), lambda i: (0, 0)),
            pl.BlockSpec((1, H), lambda i: (0, 0)),
        ],
        out_specs=pl.BlockSpec((bm, H), lambda i: (i, 0)),
        out_shape=jax.ShapeDtypeStruct((NP, H), jnp.float32),
        compiler_params=pltpu.CompilerParams(
            dimension_semantics=("parallel",)),
    )(s16, xs, dinv, w1p, b1)


def _mm_body(hd_ref, w_ref, zs_ref):
    zs_ref[0] = jnp.dot(hd_ref[...], w_ref[...],
                        preferred_element_type=jnp.float32)


def _tc_matmul_chunked(hd, w):
    bm = 512
    grid = (NP // bm, 4)
    return pl.pallas_call(
        _mm_body,
        grid=grid,
        in_specs=[
            pl.BlockSpec((bm, H), lambda i, j: (i, 0)),
            pl.BlockSpec((H, 128), lambda i, j: (0, j)),
        ],
        out_specs=pl.BlockSpec((1, bm, 128), lambda i, j: (j, i, 0)),
        out_shape=jax.ShapeDtypeStruct((4, NP, 128), jnp.float32),
        compiler_params=pltpu.CompilerParams(
            dimension_semantics=("parallel", "parallel")),
    )(hd, w)


def _ew_body(scale_out, sp_ref, zs_ref, dinv_ref, b_ref, out_ref):
    dinv = dinv_ref[...]
    z = dinv * (sp_ref[0, 0] + sp_ref[1, 0] + zs_ref[0]) + b_ref[0]
    h = jnp.where(z >= 0, z, NEG * z)
    out_ref[...] = h * dinv if scale_out else h


def _tc_ew(sp, zs, dinv, b4, scale_out):
    bm = 1024
    grid = (NP // bm, 4)
    return pl.pallas_call(
        functools.partial(_ew_body, scale_out),
        grid=grid,
        in_specs=[
            pl.BlockSpec((2, 1, bm, 128), lambda i, j: (0, j, i, 0)),
            pl.BlockSpec((1, bm, 128), lambda i, j: (j, i, 0)),
            pl.BlockSpec((bm, 1), lambda i, j: (i, 0)),
            pl.BlockSpec((1, 1, 128), lambda i, j: (j, 0, 0)),
        ],
        out_specs=pl.BlockSpec((bm, 128), lambda i, j: (i, j)),
        out_shape=jax.ShapeDtypeStruct((NP, H), jnp.float32),
        compiler_params=pltpu.CompilerParams(
            dimension_semantics=("parallel", "parallel")),
    )(sp, zs, dinv, b4)


def _fc_body(h_ref, w_ref, b_ref, out_ref):
    out_ref[...] = jnp.dot(h_ref[...], w_ref[...],
                           preferred_element_type=jnp.float32) + b_ref[...]


def _tc_fc(h, wfc, bfc2):
    bm = 1024
    grid = (NP // bm,)
    return pl.pallas_call(
        _fc_body,
        grid=grid,
        in_specs=[
            pl.BlockSpec((bm, H), lambda i: (i, 0)),
            pl.BlockSpec((H, C), lambda i: (0, 0)),
            pl.BlockSpec((1, C), lambda i: (0, 0)),
        ],
        out_specs=pl.BlockSpec((bm, C), lambda i: (i, 0)),
        out_shape=jax.ShapeDtypeStruct((NP, C), jnp.float32),
        compiler_params=pltpu.CompilerParams(
            dimension_semantics=("parallel",)),
    )(h, wfc, bfc2)


def kernel(x, edge_index, W1, b1, W2, b2, W3, b3, Wfc, bfc):
    src = edge_index[0].astype(jnp.int32).reshape(NTILES, EPT)
    dst = edge_index[1].astype(jnp.int32).reshape(NTILES, EPT)
    pad = NJ * EB - EPT
    srcp = jnp.concatenate(
        [src, jnp.zeros((NTILES, pad), jnp.int32)], axis=1
    ).reshape(NTILES, NJ, EB)
    dstp = jnp.concatenate(
        [dst, jnp.full((NTILES, pad), DUMP, jnp.int32)], axis=1
    ).reshape(NTILES, NJ, EB)

    xp = jnp.zeros((NP, 16), jnp.float32).at[:N, :F_IN].set(x)
    w1p = jnp.zeros((16, H), jnp.float32).at[:F_IN].set(W1)


    degp = _sc_degree(dstp)
    dinv, xs = _tc_prep(degp, xp)
    s16 = _sc_agg16(xs, srcp, dstp)
    hd1 = _tc_layer1(s16, xs, dinv, w1p, b1.reshape(1, H))

    zs2 = _tc_matmul_chunked(hd1, W2)
    sp2 = _sc_agg64(zs2.reshape(4 * NP, 128), srcp, dstp)
    hd2 = _tc_ew(sp2.reshape(2, 4, NP, 128), zs2, dinv,
                 b2.reshape(4, 1, 128), True)

    zs3 = _tc_matmul_chunked(hd2, W3)
    sp3 = _sc_agg64(zs3.reshape(4 * NP, 128), srcp, dstp)
    h3 = _tc_ew(sp3.reshape(2, 4, NP, 128), zs3, dinv,
                b3.reshape(4, 1, 128), False)

    out = _tc_fc(h3, Wfc, bfc.reshape(1, C))
    return out[:N]


# speedup vs baseline: 2.3989x; 1.0543x over previous
import functools

import jax
import jax.numpy as jnp
from jax import lax
from jax.experimental import pallas as pl
from jax.experimental.pallas import tpu as pltpu
from jax.experimental.pallas import tpu_sc as plsc

N = 10000
E = 160000
F_IN = 10
H = 512
C = 16

NP = 10240
DUMP = N
NTILES = 32
EPT = E // NTILES
EB = 128
NJ = 5120 // EB
RPT = NP // 16
GB = 2
NEG = 0.01

_mesh = plsc.VectorSubcoreMesh(core_axis_name="c", subcore_axis_name="s")
_sc_params = pltpu.CompilerParams(use_tc_tiling_on_sc=False)


def _fill_zeros(ref, nrows, width):
    def body(i, _):
        for l in range(width // 16):
            ref[i, pl.ds(l * 16, 16)] = jnp.zeros((16,), jnp.float32)
        return 0
    lax.fori_loop(0, nrows, body, 0)


@functools.partial(
    pl.kernel,
    out_type=jax.ShapeDtypeStruct((2, NP, 16), jnp.float32),
    mesh=_mesh,
    compiler_params=_sc_params,
    scratch_types=[
        pltpu.VMEM((NJ, EB), jnp.int32),
        pltpu.VMEM((EB, 16), jnp.float32),
        pltpu.VMEM((RPT, 16), jnp.float32),
        pltpu.VMEM_SHARED((NP, 16), jnp.float32),
    ],
)
def _sc_degree(dstp_hbm, deg_out, dst_v, ones_v, stage_v, acc_sh):
    c = lax.axis_index("c")
    s = lax.axis_index("s")
    w = c * 16 + s
    pltpu.sync_copy(dstp_hbm.at[w], dst_v)

    def fill_ones(i, _):
        ones_v[i, :] = jnp.ones((16,), jnp.float32)
        return 0
    lax.fori_loop(0, EB, fill_ones, 0)
    _fill_zeros(stage_v, RPT, 16)
    pltpu.sync_copy(stage_v, acc_sh.at[pl.ds(s * RPT, RPT)])
    plsc.subcore_barrier()

    def body(j, _):
        pltpu.sync_copy(ones_v, acc_sh.at[dst_v.at[j]], add=True)
        return 0
    lax.fori_loop(0, NJ, body, 0)
    plsc.subcore_barrier()

    pltpu.sync_copy(acc_sh.at[pl.ds(s * RPT, RPT)],
                    deg_out.at[c, pl.ds(s * RPT, RPT)])


@functools.partial(
    pl.kernel,
    out_type=jax.ShapeDtypeStruct((2, NP, 16), jnp.float32),
    mesh=_mesh,
    compiler_params=_sc_params,
    scratch_types=[
        pltpu.VMEM((NJ, EB), jnp.int32),
        pltpu.VMEM((NJ, EB), jnp.int32),
        pltpu.VMEM((EB, 16), jnp.float32),
        pltpu.VMEM((RPT, 16), jnp.float32),
        pltpu.VMEM_SHARED((NP, 16), jnp.float32),
        pltpu.VMEM_SHARED((NP, 16), jnp.float32),
        pltpu.SemaphoreType.DMA,
    ],
)
def _sc_agg16(xs_hbm, srcp_hbm, dstp_hbm, s_out,
              src_v, dst_v, rows_v, stage_v, tab_sh, acc_sh, sem):
    c = lax.axis_index("c")
    s = lax.axis_index("s")
    w = c * 16 + s
    pltpu.sync_copy(srcp_hbm.at[w], src_v)
    pltpu.sync_copy(dstp_hbm.at[w], dst_v)
    pltpu.sync_copy(xs_hbm.at[pl.ds(s * RPT, RPT)],
                    tab_sh.at[pl.ds(s * RPT, RPT)])
    _fill_zeros(stage_v, RPT, 16)
    pltpu.sync_copy(stage_v, acc_sh.at[pl.ds(s * RPT, RPT)])
    plsc.subcore_barrier()

    def body(j, _):
        pltpu.async_copy(tab_sh.at[src_v.at[j]], rows_v, sem).wait()
        pltpu.sync_copy(rows_v, acc_sh.at[dst_v.at[j]], add=True)
        return 0
    lax.fori_loop(0, NJ, body, 0)
    plsc.subcore_barrier()

    pltpu.sync_copy(acc_sh.at[pl.ds(s * RPT, RPT)],
                    s_out.at[c, pl.ds(s * RPT, RPT)])


@functools.partial(
    pl.kernel,
    out_type=jax.ShapeDtypeStruct((2, 4 * NP, 128), jnp.float32),
    mesh=_mesh,
    compiler_params=_sc_params,
    scratch_types=[
        pltpu.VMEM((NJ, EB), jnp.int32),
        pltpu.VMEM((NJ, EB), jnp.int32),
        pltpu.VMEM((EB, 64), jnp.float32),
        pltpu.VMEM((EB, 64), jnp.float32),
        pltpu.VMEM((32, 64), jnp.float32),
        pltpu.VMEM_SHARED((NP, 64), jnp.float32),
        pltpu.VMEM_SHARED((NP, 64), jnp.float32),
        pltpu.SemaphoreType.DMA,
        pltpu.SemaphoreType.DMA,
    ],
)
def _sc_agg64(zsf_hbm, srcp_hbm, dstp_hbm, s_out,
              src_v, dst_v, rows0, rows1, zero_v, tab_sh, acc_sh,
              gsem0, gsem1):
    c = lax.axis_index("c")
    s = lax.axis_index("s")
    w = c * 16 + s
    pltpu.sync_copy(srcp_hbm.at[w], src_v)
    pltpu.sync_copy(dstp_hbm.at[w], dst_v)
    _fill_zeros(zero_v, 32, 64)

    for k in range(8):
        k128, h = k // 2, k % 2
        pltpu.sync_copy(
            zsf_hbm.at[pl.ds(k128 * NP + s * RPT, RPT), pl.ds(h * 64, 64)],
            tab_sh.at[pl.ds(s * RPT, RPT)])
        for q in range(RPT // 32):
            pltpu.sync_copy(zero_v, acc_sh.at[pl.ds(s * RPT + q * 32, 32)])
        plsc.subcore_barrier()

        pltpu.async_copy(tab_sh.at[src_v.at[0]], rows0, gsem0)

        def body(g, _):
            j0 = 2 * g
            pltpu.async_copy(tab_sh.at[src_v.at[j0 + 1]], rows1, gsem1)
            pltpu.make_async_copy(
                tab_sh.at[src_v.at[j0]], rows0, gsem0).wait()
            pltpu.sync_copy(rows0, acc_sh.at[dst_v.at[j0]], add=True)

            @pl.when(g + 1 < NJ // 2)
            def _():
                pltpu.async_copy(tab_sh.at[src_v.at[j0 + 2]], rows0, gsem0)
            pltpu.make_async_copy(
                tab_sh.at[src_v.at[j0 + 1]], rows1, gsem1).wait()
            pltpu.sync_copy(rows1, acc_sh.at[dst_v.at[j0 + 1]], add=True)
            return 0
        lax.fori_loop(0, NJ // 2, body, 0)
        plsc.subcore_barrier()

        pltpu.sync_copy(
            acc_sh.at[pl.ds(s * RPT, RPT)],
            s_out.at[c, pl.ds(k128 * NP + s * RPT, RPT), pl.ds(h * 64, 64)])


def _prep_body(degp_ref, xp_ref, dinv_ref, xs_ref):
    deg = degp_ref[0, :, 0:1] + degp_ref[1, :, 0:1] + 1.0
    dinv = lax.rsqrt(deg)
    dinv_ref[...] = dinv
    xs_ref[...] = xp_ref[...] * dinv


def _tc_prep(degp, xp):
    return pl.pallas_call(
        _prep_body,
        out_shape=(
            jax.ShapeDtypeStruct((NP, 1), jnp.float32),
            jax.ShapeDtypeStruct((NP, 16), jnp.float32),
        ),
    )(degp, xp)


def _l1_body(s16_ref, xs_ref, dinv_ref, w_ref, b_ref, out_ref):
    dinv = dinv_ref[...]
    u = dinv * (s16_ref[0] + s16_ref[1] + xs_ref[...])
    z = jnp.dot(u, w_ref[...], preferred_element_type=jnp.float32)
    z = z + b_ref[...]
    h = jnp.where(z >= 0, z, NEG * z)
    out_ref[...] = (h * dinv).astype(jnp.bfloat16)


def _tc_layer1(s16, xs, dinv, w1p, b1):
    bm = 1024
    grid = (NP // bm,)
    return pl.pallas_call(
        _l1_body,
        grid=grid,
        in_specs=[
            pl.BlockSpec((2, bm, 16), lambda i: (0, i, 0)),
            pl.BlockSpec((bm, 16), lambda i: (i, 0)),
            pl.BlockSpec((bm, 1), lambda i: (i, 0)),
            pl.BlockSpec((16, H), lambda i: (0, 0)),
            pl.BlockSpec((1, H), lambda i: (0, 0)),
        ],
        out_specs=pl.BlockSpec((bm, H), lambda i: (i, 0)),
        out_shape=jax.ShapeDtypeStruct((NP, H), jnp.bfloat16),
        compiler_params=pltpu.CompilerParams(
            dimension_semantics=("parallel",)),
    )(s16, xs, dinv, w1p, b1)


def _mm_body(hd_ref, w_ref, zs_ref):
    zs_ref[0] = jnp.dot(hd_ref[...], w_ref[...],
                        preferred_element_type=jnp.float32)


def _tc_matmul_chunked(hd, w):
    bm = 512
    grid = (NP // bm, 4)
    return pl.pallas_call(
        _mm_body,
        grid=grid,
        in_specs=[
            pl.BlockSpec((bm, H), lambda i, j: (i, 0)),
            pl.BlockSpec((H, 128), lambda i, j: (0, j)),
        ],
        out_specs=pl.BlockSpec((1, bm, 128), lambda i, j: (j, i, 0)),
        out_shape=jax.ShapeDtypeStruct((4, NP, 128), jnp.float32),
        compiler_params=pltpu.CompilerParams(
            dimension_semantics=("parallel", "parallel")),
    )(hd, w)


def _ew_body(scale_out, sp_ref, zs_ref, dinv_ref, b_ref, out_ref):
    dinv = dinv_ref[...]
    z = dinv * (sp_ref[0, 0] + sp_ref[1, 0] + zs_ref[0]) + b_ref[0]
    h = jnp.where(z >= 0, z, NEG * z)
    if scale_out:
        out_ref[...] = (h * dinv).astype(jnp.bfloat16)
    else:
        out_ref[...] = h


def _tc_ew(sp, zs, dinv, b4, scale_out):
    bm = 1024
    grid = (NP // bm, 4)
    return pl.pallas_call(
        functools.partial(_ew_body, scale_out),
        grid=grid,
        in_specs=[
            pl.BlockSpec((2, 1, bm, 128), lambda i, j: (0, j, i, 0)),
            pl.BlockSpec((1, bm, 128), lambda i, j: (j, i, 0)),
            pl.BlockSpec((bm, 1), lambda i, j: (i, 0)),
            pl.BlockSpec((1, 1, 128), lambda i, j: (j, 0, 0)),
        ],
        out_specs=pl.BlockSpec((bm, 128), lambda i, j: (i, j)),
        out_shape=jax.ShapeDtypeStruct(
            (NP, H), jnp.bfloat16 if scale_out else jnp.float32),
        compiler_params=pltpu.CompilerParams(
            dimension_semantics=("parallel", "parallel")),
    )(sp, zs, dinv, b4)


def _fc_body(h_ref, w_ref, b_ref, out_ref):
    out_ref[...] = jnp.dot(h_ref[...], w_ref[...],
                           preferred_element_type=jnp.float32) + b_ref[...]


def _tc_fc(h, wfc, bfc2):
    bm = 1024
    grid = (NP // bm,)
    return pl.pallas_call(
        _fc_body,
        grid=grid,
        in_specs=[
            pl.BlockSpec((bm, H), lambda i: (i, 0)),
            pl.BlockSpec((H, C), lambda i: (0, 0)),
            pl.BlockSpec((1, C), lambda i: (0, 0)),
        ],
        out_specs=pl.BlockSpec((bm, C), lambda i: (i, 0)),
        out_shape=jax.ShapeDtypeStruct((NP, C), jnp.float32),
        compiler_params=pltpu.CompilerParams(
            dimension_semantics=("parallel",)),
    )(h, wfc, bfc2)


def kernel(x, edge_index, W1, b1, W2, b2, W3, b3, Wfc, bfc):
    src = edge_index[0].astype(jnp.int32).reshape(NTILES, EPT)
    dst = edge_index[1].astype(jnp.int32).reshape(NTILES, EPT)
    pad = NJ * EB - EPT
    srcp = jnp.concatenate(
        [src, jnp.zeros((NTILES, pad), jnp.int32)], axis=1
    ).reshape(NTILES, NJ, EB)
    dstp = jnp.concatenate(
        [dst, jnp.full((NTILES, pad), DUMP, jnp.int32)], axis=1
    ).reshape(NTILES, NJ, EB)

    xp = jnp.zeros((NP, 16), jnp.float32).at[:N, :F_IN].set(x)
    w1p = jnp.zeros((16, H), jnp.float32).at[:F_IN].set(W1)


    degp = _sc_degree(dstp)
    dinv, xs = _tc_prep(degp, xp)
    s16 = _sc_agg16(xs, srcp, dstp)
    hd1 = _tc_layer1(s16, xs, dinv, w1p, b1.reshape(1, H))

    zs2 = _tc_matmul_chunked(hd1, W2.astype(jnp.bfloat16))
    sp2 = _sc_agg64(zs2.reshape(4 * NP, 128), srcp, dstp)
    hd2 = _tc_ew(sp2.reshape(2, 4, NP, 128), zs2, dinv,
                 b2.reshape(4, 1, 128), True)

    zs3 = _tc_matmul_chunked(hd2, W3.astype(jnp.bfloat16))
    sp3 = _sc_agg64(zs3.reshape(4 * NP, 128), srcp, dstp)
    h3 = _tc_ew(sp3.reshape(2, 4, NP, 128), zs3, dinv,
                b3.reshape(4, 1, 128), False)

    out = _tc_fc(h3, Wfc, bfc.reshape(1, C))
    return out[:N]


# speedup vs baseline: 2.4422x; 1.0180x over previous
import functools

import jax
import jax.numpy as jnp
from jax import lax
from jax.experimental import pallas as pl
from jax.experimental.pallas import tpu as pltpu
from jax.experimental.pallas import tpu_sc as plsc

N = 10000
E = 160000
F_IN = 10
H = 512
C = 16

NP = 10240
DUMP = N
NTILES = 32
EPT = E // NTILES
EB = 128
NJ = 5120 // EB
RPT = NP // 16
GB = 2
NEG = 0.01

_mesh = plsc.VectorSubcoreMesh(core_axis_name="c", subcore_axis_name="s")
_sc_params = pltpu.CompilerParams(use_tc_tiling_on_sc=False)


def _fill_zeros(ref, nrows, width):
    def body(i, _):
        for l in range(width // 16):
            ref[i, pl.ds(l * 16, 16)] = jnp.zeros((16,), jnp.float32)
        return 0
    lax.fori_loop(0, nrows, body, 0)


@functools.partial(
    pl.kernel,
    out_type=jax.ShapeDtypeStruct((2, NP, 16), jnp.float32),
    mesh=_mesh,
    compiler_params=_sc_params,
    scratch_types=[
        pltpu.VMEM((NJ, EB), jnp.int32),
        pltpu.VMEM((EB, 16), jnp.float32),
        pltpu.VMEM((RPT, 16), jnp.float32),
        pltpu.VMEM_SHARED((NP, 16), jnp.float32),
    ],
)
def _sc_degree(dstp_hbm, deg_out, dst_v, ones_v, stage_v, acc_sh):
    c = lax.axis_index("c")
    s = lax.axis_index("s")
    w = c * 16 + s
    pltpu.sync_copy(dstp_hbm.at[w], dst_v)

    def fill_ones(i, _):
        ones_v[i, :] = jnp.ones((16,), jnp.float32)
        return 0
    lax.fori_loop(0, EB, fill_ones, 0)
    _fill_zeros(stage_v, RPT, 16)
    pltpu.sync_copy(stage_v, acc_sh.at[pl.ds(s * RPT, RPT)])
    plsc.subcore_barrier()

    def body(j, _):
        pltpu.sync_copy(ones_v, acc_sh.at[dst_v.at[j]], add=True)
        return 0
    lax.fori_loop(0, NJ, body, 0)
    plsc.subcore_barrier()

    pltpu.sync_copy(acc_sh.at[pl.ds(s * RPT, RPT)],
                    deg_out.at[c, pl.ds(s * RPT, RPT)])


@functools.partial(
    pl.kernel,
    out_type=jax.ShapeDtypeStruct((2, NP, 16), jnp.float32),
    mesh=_mesh,
    compiler_params=_sc_params,
    scratch_types=[
        pltpu.VMEM((NJ, EB), jnp.int32),
        pltpu.VMEM((NJ, EB), jnp.int32),
        pltpu.VMEM((EB, 16), jnp.float32),
        pltpu.VMEM((RPT, 16), jnp.float32),
        pltpu.VMEM_SHARED((NP, 16), jnp.float32),
        pltpu.VMEM_SHARED((NP, 16), jnp.float32),
        pltpu.SemaphoreType.DMA,
    ],
)
def _sc_agg16(xs_hbm, srcp_hbm, dstp_hbm, s_out,
              src_v, dst_v, rows_v, stage_v, tab_sh, acc_sh, sem):
    c = lax.axis_index("c")
    s = lax.axis_index("s")
    w = c * 16 + s
    pltpu.sync_copy(srcp_hbm.at[w], src_v)
    pltpu.sync_copy(dstp_hbm.at[w], dst_v)
    pltpu.sync_copy(xs_hbm.at[pl.ds(s * RPT, RPT)],
                    tab_sh.at[pl.ds(s * RPT, RPT)])
    _fill_zeros(stage_v, RPT, 16)
    pltpu.sync_copy(stage_v, acc_sh.at[pl.ds(s * RPT, RPT)])
    plsc.subcore_barrier()

    def body(j, _):
        pltpu.async_copy(tab_sh.at[src_v.at[j]], rows_v, sem).wait()
        pltpu.sync_copy(rows_v, acc_sh.at[dst_v.at[j]], add=True)
        return 0
    lax.fori_loop(0, NJ, body, 0)
    plsc.subcore_barrier()

    pltpu.sync_copy(acc_sh.at[pl.ds(s * RPT, RPT)],
                    s_out.at[c, pl.ds(s * RPT, RPT)])


@functools.partial(
    pl.kernel,
    out_type=jax.ShapeDtypeStruct((2, 4 * NP, 128), jnp.float32),
    mesh=_mesh,
    compiler_params=_sc_params,
    scratch_types=[
        pltpu.VMEM((NJ, EB), jnp.int32),
        pltpu.VMEM((NJ, EB), jnp.int32),
        pltpu.VMEM((EB, 64), jnp.float32),
        pltpu.VMEM((EB, 64), jnp.float32),
        pltpu.VMEM((160, 64), jnp.float32),
        pltpu.VMEM_SHARED((NP, 64), jnp.float32),
        pltpu.VMEM_SHARED((NP, 64), jnp.float32),
        pltpu.SemaphoreType.DMA,
        pltpu.SemaphoreType.DMA,
    ],
)
def _sc_agg64(zsf_hbm, srcp_hbm, dstp_hbm, s_out,
              src_v, dst_v, rows0, rows1, zero_v, tb, acc_sh,
              gsem0, gsem1):
    c = lax.axis_index("c")
    s = lax.axis_index("s")
    w = c * 16 + s
    pltpu.sync_copy(srcp_hbm.at[w], src_v)
    pltpu.sync_copy(dstp_hbm.at[w], dst_v)
    _fill_zeros(zero_v, 160, 64)

    for k in range(8):
        k128, h = k // 2, k % 2
        pltpu.sync_copy(
            zsf_hbm.at[pl.ds(k128 * NP + s * RPT, RPT), pl.ds(h * 64, 64)],
            tb.at[pl.ds(s * RPT, RPT)])
        for q in range(RPT // 160):
            pltpu.sync_copy(zero_v, acc_sh.at[pl.ds(s * RPT + q * 160, 160)])
        plsc.subcore_barrier()

        pltpu.async_copy(tb.at[src_v.at[0]], rows0, gsem0)

        def body(g, _):
            j0 = 2 * g
            pltpu.async_copy(tb.at[src_v.at[j0 + 1]], rows1, gsem1)
            pltpu.make_async_copy(tb.at[src_v.at[j0]], rows0, gsem0).wait()
            pltpu.sync_copy(rows0, acc_sh.at[dst_v.at[j0]], add=True)

            @pl.when(g + 1 < NJ // 2)
            def _():
                pltpu.async_copy(tb.at[src_v.at[j0 + 2]], rows0, gsem0)
            pltpu.make_async_copy(
                tb.at[src_v.at[j0 + 1]], rows1, gsem1).wait()
            pltpu.sync_copy(rows1, acc_sh.at[dst_v.at[j0 + 1]], add=True)
            return 0
        lax.fori_loop(0, NJ // 2, body, 0)
        plsc.subcore_barrier()

        pltpu.sync_copy(
            acc_sh.at[pl.ds(s * RPT, RPT)],
            s_out.at[c, pl.ds(k128 * NP + s * RPT, RPT), pl.ds(h * 64, 64)])


def _prep_body(degp_ref, xp_ref, dinv_ref, xs_ref):
    deg = degp_ref[0, :, 0:1] + degp_ref[1, :, 0:1] + 1.0
    dinv = lax.rsqrt(deg)
    dinv_ref[...] = dinv
    xs_ref[...] = xp_ref[...] * dinv


def _tc_prep(degp, xp):
    return pl.pallas_call(
        _prep_body,
        out_shape=(
            jax.ShapeDtypeStruct((NP, 1), jnp.float32),
            jax.ShapeDtypeStruct((NP, 16), jnp.float32),
        ),
    )(degp, xp)


def _l1_body(s16_ref, xs_ref, dinv_ref, w_ref, b_ref, out_ref):
    dinv = dinv_ref[...]
    u = dinv * (s16_ref[0] + s16_ref[1] + xs_ref[...])
    z = jnp.dot(u, w_ref[...], preferred_element_type=jnp.float32)
    z = z + b_ref[...]
    h = jnp.where(z >= 0, z, NEG * z)
    out_ref[...] = (h * dinv).astype(jnp.bfloat16)


def _tc_layer1(s16, xs, dinv, w1p, b1):
    bm = 1024
    grid = (NP // bm,)
    return pl.pallas_call(
        _l1_body,
        grid=grid,
        in_specs=[
            pl.BlockSpec((2, bm, 16), lambda i: (0, i, 0)),
            pl.BlockSpec((bm, 16), lambda i: (i, 0)),
            pl.BlockSpec((bm, 1), lambda i: (i, 0)),
            pl.BlockSpec((16, H), lambda i: (0, 0)),
            pl.BlockSpec((1, H), lambda i: (0, 0)),
        ],
        out_specs=pl.BlockSpec((bm, H), lambda i: (i, 0)),
        out_shape=jax.ShapeDtypeStruct((NP, H), jnp.bfloat16),
        compiler_params=pltpu.CompilerParams(
            dimension_semantics=("parallel",)),
    )(s16, xs, dinv, w1p, b1)


def _mm_body(hd_ref, w_ref, zs_ref):
    zs_ref[0] = jnp.dot(hd_ref[...], w_ref[...],
                        preferred_element_type=jnp.float32)


def _tc_matmul_chunked(hd, w):
    bm = 512
    grid = (NP // bm, 4)
    return pl.pallas_call(
        _mm_body,
        grid=grid,
        in_specs=[
            pl.BlockSpec((bm, H), lambda i, j: (i, 0)),
            pl.BlockSpec((H, 128), lambda i, j: (0, j)),
        ],
        out_specs=pl.BlockSpec((1, bm, 128), lambda i, j: (j, i, 0)),
        out_shape=jax.ShapeDtypeStruct((4, NP, 128), jnp.float32),
        compiler_params=pltpu.CompilerParams(
            dimension_semantics=("parallel", "parallel")),
    )(hd, w)


def _ew_body(scale_out, sp_ref, zs_ref, dinv_ref, b_ref, out_ref):
    dinv = dinv_ref[...]
    z = dinv * (sp_ref[0, 0] + sp_ref[1, 0] + zs_ref[0]) + b_ref[0]
    h = jnp.where(z >= 0, z, NEG * z)
    if scale_out:
        out_ref[...] = (h * dinv).astype(jnp.bfloat16)
    else:
        out_ref[...] = h


def _tc_ew(sp, zs, dinv, b4, scale_out):
    bm = 1024
    grid = (NP // bm, 4)
    return pl.pallas_call(
        functools.partial(_ew_body, scale_out),
        grid=grid,
        in_specs=[
            pl.BlockSpec((2, 1, bm, 128), lambda i, j: (0, j, i, 0)),
            pl.BlockSpec((1, bm, 128), lambda i, j: (j, i, 0)),
            pl.BlockSpec((bm, 1), lambda i, j: (i, 0)),
            pl.BlockSpec((1, 1, 128), lambda i, j: (j, 0, 0)),
        ],
        out_specs=pl.BlockSpec((bm, 128), lambda i, j: (i, j)),
        out_shape=jax.ShapeDtypeStruct(
            (NP, H), jnp.bfloat16 if scale_out else jnp.float32),
        compiler_params=pltpu.CompilerParams(
            dimension_semantics=("parallel", "parallel")),
    )(sp, zs, dinv, b4)


def _fc_body(h_ref, w_ref, b_ref, out_ref):
    out_ref[...] = jnp.dot(h_ref[...], w_ref[...],
                           preferred_element_type=jnp.float32) + b_ref[...]


def _tc_fc(h, wfc, bfc2):
    bm = 1024
    grid = (NP // bm,)
    return pl.pallas_call(
        _fc_body,
        grid=grid,
        in_specs=[
            pl.BlockSpec((bm, H), lambda i: (i, 0)),
            pl.BlockSpec((H, C), lambda i: (0, 0)),
            pl.BlockSpec((1, C), lambda i: (0, 0)),
        ],
        out_specs=pl.BlockSpec((bm, C), lambda i: (i, 0)),
        out_shape=jax.ShapeDtypeStruct((NP, C), jnp.float32),
        compiler_params=pltpu.CompilerParams(
            dimension_semantics=("parallel",)),
    )(h, wfc, bfc2)


def kernel(x, edge_index, W1, b1, W2, b2, W3, b3, Wfc, bfc):
    src = edge_index[0].astype(jnp.int32).reshape(NTILES, EPT)
    dst = edge_index[1].astype(jnp.int32).reshape(NTILES, EPT)
    pad = NJ * EB - EPT
    srcp = jnp.concatenate(
        [src, jnp.zeros((NTILES, pad), jnp.int32)], axis=1
    ).reshape(NTILES, NJ, EB)
    dstp = jnp.concatenate(
        [dst, jnp.full((NTILES, pad), DUMP, jnp.int32)], axis=1
    ).reshape(NTILES, NJ, EB)

    xp = jnp.zeros((NP, 16), jnp.float32).at[:N, :F_IN].set(x)
    w1p = jnp.zeros((16, H), jnp.float32).at[:F_IN].set(W1)


    degp = _sc_degree(dstp)
    dinv, xs = _tc_prep(degp, xp)
    s16 = _sc_agg16(xs, srcp, dstp)
    hd1 = _tc_layer1(s16, xs, dinv, w1p, b1.reshape(1, H))

    zs2 = _tc_matmul_chunked(hd1, W2.astype(jnp.bfloat16))
    sp2 = _sc_agg64(zs2.reshape(4 * NP, 128), srcp, dstp)
    hd2 = _tc_ew(sp2.reshape(2, 4, NP, 128), zs2, dinv,
                 b2.reshape(4, 1, 128), True)

    zs3 = _tc_matmul_chunked(hd2, W3.astype(jnp.bfloat16))
    sp3 = _sc_agg64(zs3.reshape(4 * NP, 128), srcp, dstp)
    h3 = _tc_ew(sp3.reshape(2, 4, NP, 128), zs3, dinv,
                b3.reshape(4, 1, 128), False)

    out = _tc_fc(h3, Wfc, bfc.reshape(1, C))
    return out[:N]
